# baseline (speedup 1.0000x reference)
import functools

import jax
import jax.numpy as jnp
from jax import lax
from jax.experimental import pallas as pl
from jax.experimental.pallas import tpu as pltpu
from jax.experimental.pallas import tpu_sc as plsc

_TB = 1024


def _vq_idx_body(d_ref, z_ref, idx_ref, d2_ref, dn_ref):
    @pl.when(pl.program_id(0) == 0)
    def _prologue():
        d = d_ref[...]
        d2_ref[...] = -2.0 * d
        dn_ref[...] = jnp.sum(d * d, axis=1, keepdims=True)

    zt = z_ref[...].T
    dots2 = lax.dot_general(d2_ref[...], zt, (((1,), (0,)), ((), ())),
                            preferred_element_type=jnp.float32)
    zn = jnp.sum(zt * zt, axis=0)
    dist = (dots2 + dn_ref[...]) + zn[None, :]
    idx_ref[...] = jnp.argmin(dist, axis=0).astype(jnp.int32)


def _sc_gather(dict_hbm, idx_hbm, emb_hbm, emb_pt_hbm, idx_v, rows_v, sem):
    nc = 2
    wid = lax.axis_index("s") * nc + lax.axis_index("c")
    base = wid * 512
    for j in range(4):
        pltpu.sync_copy(idx_hbm.at[pl.ds(base + j * 128, 128)], idx_v.at[j])
    copies = []
    for j in range(4):
        copies.append(pltpu.async_copy(
            dict_hbm.at[idx_v.at[j]],
            rows_v.at[pl.ds(j * 128, 128)], sem))
    for cp in copies:
        cp.wait()
    pltpu.sync_copy(rows_v, emb_hbm.at[pl.ds(base, 512)])
    pltpu.sync_copy(rows_v, emb_pt_hbm.at[pl.ds(base, 512)])


@jax.jit
def kernel(inputs, dictionary):
    n, c, h, w = inputs.shape
    nt = n * h * w
    z = jnp.transpose(inputs, (0, 2, 3, 1)).reshape(nt, c)
    nb = nt // _TB
    idx = pl.pallas_call(
        _vq_idx_body,
        grid=(nb,),
        in_specs=[
            pl.BlockSpec((1024, c), lambda i: (0, 0)),
            pl.BlockSpec((_TB, c), lambda i: (i, 0)),
        ],
        out_specs=pl.BlockSpec((_TB,), lambda i: (i,)),
        out_shape=jax.ShapeDtypeStruct((nt,), jnp.int32),
        scratch_shapes=[
            pltpu.VMEM((1024, c), jnp.float32),
            pltpu.VMEM((1024, 1), jnp.float32),
        ],
    )(dictionary, z)

    mesh = plsc.VectorSubcoreMesh(core_axis_name="c", subcore_axis_name="s")
    gather = pl.kernel(
        _sc_gather, mesh=mesh,
        out_type=[
            jax.ShapeDtypeStruct((nt, c), jnp.float32),
            jax.ShapeDtypeStruct((nt, c), jnp.float32),
        ],
        scratch_types=[
            pltpu.VMEM((4, 128), jnp.int32),
            pltpu.VMEM((512, c), jnp.float32),
            pltpu.SemaphoreType.DMA,
        ],
        compiler_params=pltpu.CompilerParams(use_tc_tiling_on_sc=False),
    )
    emb, emb_pt = gather(dictionary, idx)

    emb = jnp.transpose(emb.reshape(n, h, w, c), (0, 3, 1, 2))
    emb_pt = jnp.transpose(emb_pt.reshape(n, h, w, c), (0, 3, 1, 2))
    return emb, emb_pt, idx.reshape(n, h, w)


# speedup vs baseline: 1.9876x; 1.9876x over previous
import jax
import jax.numpy as jnp
from jax import lax
from jax.experimental import pallas as pl
from jax.experimental.pallas import tpu as pltpu

_TB = 1024


def _vq_body(d_ref, z_ref, emb_ref, emb_pt_ref, idx_ref, d2_ref, dn_ref):
    @pl.when(pl.program_id(0) == 0)
    def _prologue():
        d = d_ref[...]
        d2_ref[...] = -2.0 * d
        dn_ref[...] = jnp.sum(d * d, axis=1, keepdims=True)

    zt = z_ref[...].T
    dots2 = lax.dot_general(d2_ref[...], zt, (((1,), (0,)), ((), ())),
                            preferred_element_type=jnp.float32)
    zn = jnp.sum(zt * zt, axis=0)
    dist = (dots2 + dn_ref[...]) + zn[None, :]
    idx = jnp.argmin(dist, axis=0).astype(jnp.int32)
    iota = lax.broadcasted_iota(jnp.int32, (1024, _TB), 0)
    onehot = (iota == idx[None, :]).astype(jnp.float32)
    embt = lax.dot_general(d_ref[...], onehot, (((0,), (0,)), ((), ())),
                           preferred_element_type=jnp.float32)
    emb = embt.T
    emb_ref[...] = emb
    emb_pt_ref[...] = emb
    idx_ref[...] = idx


@jax.jit
def kernel(inputs, dictionary):
    n, c, h, w = inputs.shape
    nt = n * h * w
    z = jnp.transpose(inputs, (0, 2, 3, 1)).reshape(nt, c)
    nb = nt // _TB
    emb, emb_pt, idx = pl.pallas_call(
        _vq_body,
        grid=(nb,),
        in_specs=[
            pl.BlockSpec((1024, c), lambda i: (0, 0)),
            pl.BlockSpec((_TB, c), lambda i: (i, 0)),
        ],
        out_specs=[
            pl.BlockSpec((_TB, c), lambda i: (i, 0)),
            pl.BlockSpec((_TB, c), lambda i: (i, 0)),
            pl.BlockSpec((_TB,), lambda i: (i,)),
        ],
        out_shape=[
            jax.ShapeDtypeStruct((nt, c), jnp.float32),
            jax.ShapeDtypeStruct((nt, c), jnp.float32),
            jax.ShapeDtypeStruct((nt,), jnp.int32),
        ],
        scratch_shapes=[
            pltpu.VMEM((1024, c), jnp.float32),
            pltpu.VMEM((1024, 1), jnp.float32),
        ],
    )(dictionary, z)
    emb = jnp.transpose(emb.reshape(n, h, w, c), (0, 3, 1, 2))
    emb_pt = jnp.transpose(emb_pt.reshape(n, h, w, c), (0, 3, 1, 2))
    idx = idx.reshape(n, h, w)
    return emb, emb_pt, idx


# speedup vs baseline: 2.1544x; 1.0839x over previous
import jax
import jax.numpy as jnp
from jax import lax
from jax.experimental import pallas as pl
from jax.experimental.pallas import tpu as pltpu

_TB = 2048


def _vq_body(d_ref, z_ref, emb_ref, emb_pt_ref, idx_ref, d2_ref, dn_ref):
    @pl.when(pl.program_id(0) == 0)
    def _prologue():
        d = d_ref[...]
        d2_ref[...] = -2.0 * d
        dn_ref[...] = jnp.sum(d * d, axis=1, keepdims=True)

    zt = z_ref[...].T
    dots2 = lax.dot_general(d2_ref[...], zt, (((1,), (0,)), ((), ())),
                            preferred_element_type=jnp.float32)
    zn = jnp.sum(zt * zt, axis=0)
    dist = (dots2 + dn_ref[...]) + zn[None, :]
    idx = jnp.argmin(dist, axis=0).astype(jnp.int32)
    iota = lax.broadcasted_iota(jnp.int32, (1024, _TB), 0)
    onehot = (iota == idx[None, :]).astype(jnp.float32)
    embt = lax.dot_general(d_ref[...], onehot, (((0,), (0,)), ((), ())),
                           preferred_element_type=jnp.float32)
    emb = embt.T
    emb_ref[...] = emb
    emb_pt_ref[...] = emb
    idx_ref[...] = idx


@jax.jit
def kernel(inputs, dictionary):
    n, c, h, w = inputs.shape
    nt = n * h * w
    z = jnp.transpose(inputs, (0, 2, 3, 1)).reshape(nt, c)
    nb = nt // _TB
    emb, emb_pt, idx = pl.pallas_call(
        _vq_body,
        grid=(nb,),
        in_specs=[
            pl.BlockSpec((1024, c), lambda i: (0, 0)),
            pl.BlockSpec((_TB, c), lambda i: (i, 0)),
        ],
        out_specs=[
            pl.BlockSpec((_TB, c), lambda i: (i, 0)),
            pl.BlockSpec((_TB, c), lambda i: (i, 0)),
            pl.BlockSpec((_TB,), lambda i: (i,)),
        ],
        out_shape=[
            jax.ShapeDtypeStruct((nt, c), jnp.float32),
            jax.ShapeDtypeStruct((nt, c), jnp.float32),
            jax.ShapeDtypeStruct((nt,), jnp.int32),
        ],
        scratch_shapes=[
            pltpu.VMEM((1024, c), jnp.float32),
            pltpu.VMEM((1024, 1), jnp.float32),
        ],
    )(dictionary, z)
    emb = jnp.transpose(emb.reshape(n, h, w, c), (0, 3, 1, 2))
    emb_pt = jnp.transpose(emb_pt.reshape(n, h, w, c), (0, 3, 1, 2))
    idx = idx.reshape(n, h, w)
    return emb, emb_pt, idx
